# baseline (speedup 1.0000x reference)
import functools

import jax
import jax.numpy as jnp
from jax import lax
from jax.experimental import pallas as pl
from jax.experimental.pallas import tpu as pltpu
from jax.experimental.pallas import tpu_sc as plsc

D = 32
L = 16
CHW = 128


@functools.lru_cache(maxsize=None)
def _build(B):
    info = plsc.get_sparse_core_info()
    NC, NS = info.num_cores, info.num_subcores
    NW = NC * NS
    bpw = B // NW
    CH = bpw // CHW
    NG = bpw // L

    mesh = plsc.VectorSubcoreMesh(core_axis_name="c", subcore_axis_name="s")

    @functools.partial(
        pl.kernel,
        mesh=mesh,
        out_type=jax.ShapeDtypeStruct((B,), jnp.float32),
        compiler_params=pltpu.CompilerParams(
            needs_layout_passes=False, use_tc_tiling_on_sc=False),
        scratch_types=[
            pltpu.VMEM((CH, CHW), jnp.int32),
            pltpu.VMEM((CH, CHW), jnp.int32),
            pltpu.VMEM((bpw, D), jnp.float32),
            pltpu.VMEM((bpw, D), jnp.float32),
            pltpu.VMEM((D,), jnp.float32),
            pltpu.VMEM((L,), jnp.float32),
            pltpu.VMEM((bpw,), jnp.float32),
            pltpu.SemaphoreType.DMA,
        ],
    )
    def sc_kernel(uidx_h, iidx_h, utab_h, itab_h, w_h, b_h, out_h,
                  uix, iix, urows, irows, wv, bv, outv, sem):
        wid = lax.axis_index("s") * NC + lax.axis_index("c")
        base = wid * bpw

        pltpu.sync_copy(uidx_h.at[wid], uix)
        pltpu.sync_copy(iidx_h.at[wid], iix)
        pltpu.sync_copy(w_h, wv)
        pltpu.sync_copy(b_h, bv)

        copies = []
        for c in range(CH):
            copies.append(pltpu.async_copy(
                utab_h.at[uix.at[c]], urows.at[pl.ds(c * CHW, CHW)], sem))
            copies.append(pltpu.async_copy(
                itab_h.at[iix.at[c]], irows.at[pl.ds(c * CHW, CHW)], sem))
        for cp in copies:
            cp.wait()

        bvec = bv[...]
        lane = lax.iota(jnp.int32, L)
        w_lo = wv[pl.ds(0, L)]
        w_hi = wv[pl.ds(L, L)]

        def group(g, carry):
            rows = g * L + lane
            acc = bvec
            for d in range(D):
                dv = jnp.full((L,), d, dtype=jnp.int32)
                u_d = plsc.load_gather(urows, [rows, dv])
                i_d = plsc.load_gather(irows, [rows, dv])
                w_d = w_lo[d] if d < L else w_hi[d - L]
                acc = acc + u_d * i_d * w_d
            outv[pl.ds(g * L, L)] = acc
            return carry

        lax.fori_loop(0, NG, group, 0)

        pltpu.sync_copy(outv, out_h.at[pl.ds(base, bpw)])

    return sc_kernel, NW, CH


def kernel(user_idx, item_idx, user_table, item_table, W, b):
    B = user_idx.shape[0]
    sc_kernel, NW, CH = _build(B)
    uidx3 = user_idx.reshape(NW, CH, CHW)
    iidx3 = item_idx.reshape(NW, CH, CHW)
    wflat = W.reshape(-1)
    bvec = jnp.broadcast_to(b, (L,))
    return sc_kernel(uidx3, iidx3, user_table, item_table, wflat, bvec)


# speedup vs baseline: 3.7355x; 3.7355x over previous
import functools

import jax
import jax.numpy as jnp
from jax import lax
from jax.experimental import pallas as pl
from jax.experimental.pallas import tpu as pltpu
from jax.experimental.pallas import tpu_sc as plsc

D = 32
L = 16
TW = 128


@functools.lru_cache(maxsize=None)
def _build(B):
    info = plsc.get_sparse_core_info()
    NC, NS = info.num_cores, info.num_subcores
    NW = NC * NS
    bpw = B // NW
    NG = bpw // L

    mesh = plsc.VectorSubcoreMesh(core_axis_name="c", subcore_axis_name="s")

    @functools.partial(
        pl.kernel,
        mesh=mesh,
        out_type=jax.ShapeDtypeStruct((B,), jnp.float32),
        compiler_params=pltpu.CompilerParams(
            needs_layout_passes=False, disable_bounds_checks=True),
        scratch_types=[
            pltpu.VMEM((bpw,), jnp.int32),
            pltpu.VMEM((bpw,), jnp.int32),
            pltpu.VMEM((L, D, TW), jnp.float32),
            pltpu.VMEM((D, L), jnp.float32),
            pltpu.VMEM((D,), jnp.float32),
            pltpu.VMEM((L,), jnp.float32),
            pltpu.VMEM((bpw,), jnp.float32),
            pltpu.SemaphoreType.DMA,
        ],
    )
    def sc_kernel(uidx_h, iidx_h, utabT_h, itabT_h, w_h, b_h, out_h,
                  uixv, iixv, blk, stage, wv, bv, outv, sem):
        wid = lax.axis_index("s") * NC + lax.axis_index("c")
        base = wid * bpw

        pltpu.sync_copy(uidx_h.at[pl.ds(base, bpw)], uixv)
        pltpu.sync_copy(iidx_h.at[pl.ds(base, bpw)], iixv)
        pltpu.sync_copy(w_h, wv)
        pltpu.sync_copy(b_h, bv)

        w_lo = wv[pl.ds(0, L)]
        w_hi = wv[pl.ds(L, L)]
        bvec = bv[...]
        lane = lax.iota(jnp.int32, L)

        def fetch_blocks(tab_h, cs):
            for j in range(L):
                off = pl.multiple_of(cs[j], TW)
                pltpu.async_copy(tab_h.at[:, pl.ds(off, TW)], blk.at[j], sem)
            for j in range(L):
                pltpu.make_async_copy(
                    tab_h.at[:, pl.ds(0, TW)], blk.at[j], sem).wait()

        def group(g, carry):
            uvec = uixv[pl.ds(g * L, L)]
            fetch_blocks(utabT_h, uvec & -TW)
            uph = uvec & (TW - 1)
            for d in range(D):
                dv = jnp.full((L,), d, dtype=jnp.int32)
                w_d = w_lo[d] if d < L else w_hi[d - L]
                stage[d, :] = plsc.load_gather(blk, [lane, dv, uph]) * w_d

            ivec = iixv[pl.ds(g * L, L)]
            fetch_blocks(itabT_h, ivec & -TW)
            iph = ivec & (TW - 1)
            acc = bvec
            for d in range(D):
                dv = jnp.full((L,), d, dtype=jnp.int32)
                acc = acc + stage[d, :] * plsc.load_gather(blk, [lane, dv, iph])
            outv[pl.ds(g * L, L)] = acc
            return carry

        lax.fori_loop(0, NG, group, 0)

        pltpu.sync_copy(outv, out_h.at[pl.ds(base, bpw)])

    return sc_kernel


def kernel(user_idx, item_idx, user_table, item_table, W, b):
    B = user_idx.shape[0]
    sc_kernel = _build(B)
    return sc_kernel(user_idx, item_idx, user_table.T, item_table.T,
                     W.reshape(-1), jnp.broadcast_to(b, (L,)))
